# baseline (speedup 1.0000x reference)
import jax
import jax.numpy as jnp
from jax.experimental import pallas as pl
from jax.experimental.pallas import tpu as pltpu

B = 8192
D = 128
G = 128
H = 4
NEG = 0.2

BB = 512
HG = H * G
NL = 16
EXT = HG + 2 * NL


def _att_mats(att_src, att_dst):
    c = jnp.arange(3 * H)
    mask_src = (c[None, :] % H == jnp.arange(H)[:, None]).astype(jnp.float32)
    Msrc = (att_src[:, :, None] * mask_src[:, None, :]).reshape(HG, 3 * H)
    Msrc = jnp.pad(Msrc, ((0, 0), (0, NL - 3 * H)))
    Mdsts = []
    for i in range(3):
        mask_i = (c[None, :] == (i * H + jnp.arange(H)[:, None])).astype(jnp.float32)
        Mi = (att_dst[:, :, None] * mask_i[:, None, :]).reshape(HG, 3 * H)
        Mdsts.append(jnp.pad(Mi, ((0, 0), (0, NL - 3 * H))))
    return Msrc, Mdsts


def _ext_weights(W, att_src, att_dst):
    Msrc, Mdsts = _att_mats(att_src, att_dst)
    WMs = W @ Msrc
    return [jnp.concatenate([W, WMs, W @ Mdsts[i]], axis=1) for i in range(3)]


def _dotT(A, X, dA, dX):
    return jax.lax.dot_general(A, X, (((dA,), (dX,)), ((), ())),
                               preferred_element_type=jnp.float32)


def _gat_layer(xTs, xdim, Wrefs, b_ref):
    he = [_dotT(Wrefs[i][:], xTs[i], 0, xdim) for i in range(3)]
    h = [he[i][0:HG, :] for i in range(3)]
    dl = (he[0][HG + NL:, :] + he[1][HG + NL:, :]
          + he[2][HG + NL:, :])
    ex = []
    for i in range(3):
        L = he[i][HG:HG + NL, :] + dl
        ex.append(jnp.where(L > 0, L, NEG * L))
    m = jnp.maximum(jnp.maximum(ex[0], ex[1]), ex[2])
    ex = [jnp.exp(v - m) for v in ex]
    inv = 1.0 / (ex[0] + ex[1] + ex[2] + 1e-16)
    al = [v * inv for v in ex]
    accs = [None, None, None]
    for k in range(H):
        hcs = [h[i][k * G:(k + 1) * G, :] for i in range(3)]
        for j in range(3):
            c = j * H + k
            contrib = (al[0][c:c + 1, :] * hcs[0]
                       + al[1][c:c + 1, :] * hcs[1]
                       + al[2][c:c + 1, :] * hcs[2])
            accs[j] = contrib if accs[j] is None else accs[j] + contrib
    return [a * (1.0 / H) + b_ref[:] for a in accs]


def _fused_kernel(t_ref, a_ref, v_ref,
                  W0a_ref, W0b_ref, W0c_ref, b0_ref,
                  W1a_ref, W1b_ref, W1c_ref, b1_ref,
                  Wout_ref, bout_ref, out_ref):
    xs = [t_ref[:], a_ref[:], v_ref[:]]
    ys = _gat_layer(xs, 1, [W0a_ref, W0b_ref, W0c_ref], b0_ref)
    ys = [jnp.maximum(y, 0.0) for y in ys]
    zs = _gat_layer(ys, 0, [W1a_ref, W1b_ref, W1c_ref], b1_ref)
    zs = [jnp.maximum(z, 0.0) for z in zs]
    pooled = (zs[0] + zs[1] + zs[2]) * (1.0 / 3.0)
    out_ref[:] = _dotT(pooled, Wout_ref[:], 0, 0) + bout_ref[:]


def kernel(text_features, audio_features, video_features, type_emb,
           W0, att_src0, att_dst0, b0, W1, att_src1, att_dst1, b1, Wout, bout):
    W0e = _ext_weights(W0, att_src0, att_dst0)
    W1e = _ext_weights(W1, att_src1, att_dst1)
    b0_col = jnp.broadcast_to(b0.reshape(G, 1), (G, 128))
    b1_col = jnp.broadcast_to(b1.reshape(G, 1), (G, 128))
    bout_row = bout.reshape(1, D)

    te_bias = [jnp.broadcast_to((type_emb[i:i + 1, :] @ W0e[i]).reshape(EXT, 1),
                                (EXT, 128)) for i in range(3)]

    grid = (B // BB,)
    feat_spec = pl.BlockSpec((BB, D), lambda i: (i, 0))
    full = lambda shape: pl.BlockSpec(shape, lambda i: (0,) * len(shape))

    def body(t_ref, a_ref, v_ref,
             W0a, W0b, W0c, te0, te1, te2, b0r,
             W1a, W1b, W1c, b1r, Woutr, boutr, out_ref):
        xs = [t_ref[:], a_ref[:], v_ref[:]]
        Wr = [W0a, W0b, W0c]
        ter = [te0, te1, te2]
        he = [_dotT(Wr[i][:], xs[i], 0, 1) + ter[i][:, 0:1] for i in range(3)]
        h = [he[i][0:HG, :] for i in range(3)]
        dl = he[0][HG + NL:, :] + he[1][HG + NL:, :] + he[2][HG + NL:, :]
        ex = []
        for i in range(3):
            L = he[i][HG:HG + NL, :] + dl
            ex.append(jnp.where(L > 0, L, NEG * L))
        m = jnp.maximum(jnp.maximum(ex[0], ex[1]), ex[2])
        ex = [jnp.exp(v - m) for v in ex]
        inv = 1.0 / (ex[0] + ex[1] + ex[2] + 1e-16)
        al = [v * inv for v in ex]
        accs = [None, None, None]
        for k in range(H):
            hcs = [h[i][k * G:(k + 1) * G, :] for i in range(3)]
            for j in range(3):
                c = j * H + k
                contrib = (al[0][c:c + 1, :] * hcs[0]
                           + al[1][c:c + 1, :] * hcs[1]
                           + al[2][c:c + 1, :] * hcs[2])
                accs[j] = contrib if accs[j] is None else accs[j] + contrib
        ys = [jnp.maximum(a * (1.0 / H) + b0r[:, 0:1], 0.0) for a in accs]
        zs = _gat_layer(ys, 0, [W1a, W1b, W1c], b1r[:, 0:1])
        zs = [jnp.maximum(z, 0.0) for z in zs]
        pooled = (zs[0] + zs[1] + zs[2]) * (1.0 / 3.0)
        out_ref[:] = _dotT(pooled, Woutr[:], 0, 0) + boutr[:]

    return pl.pallas_call(
        body,
        grid=grid,
        in_specs=[
            feat_spec, feat_spec, feat_spec,
            full((D, EXT)), full((D, EXT)), full((D, EXT)),
            full((EXT, 128)), full((EXT, 128)), full((EXT, 128)), full((G, 128)),
            full((G, EXT)), full((G, EXT)), full((G, EXT)), full((G, 128)),
            full((G, D)), full((1, D)),
        ],
        out_specs=pl.BlockSpec((BB, D), lambda i: (i, 0)),
        out_shape=jax.ShapeDtypeStruct((B, D), jnp.float32),
        compiler_params=pltpu.CompilerParams(
            dimension_semantics=("parallel",)),
    )(text_features, audio_features, video_features,
      W0e[0], W0e[1], W0e[2], te_bias[0], te_bias[1], te_bias[2], b0_col,
      W1e[0], W1e[1], W1e[2], b1_col,
      Wout, bout_row)


# speedup vs baseline: 1.2123x; 1.2123x over previous
import jax
import jax.numpy as jnp
from jax.experimental import pallas as pl
from jax.experimental.pallas import tpu as pltpu

B = 8192
D = 128
G = 128
H = 4
NEG = 0.2

BB = 1024
HG = H * G
NL = 16
EXT = HG + 2 * NL


def _att_mats(att_src, att_dst):
    c = jnp.arange(3 * H)
    mask_src = (c[None, :] % H == jnp.arange(H)[:, None]).astype(jnp.float32)
    Msrc = (att_src[:, :, None] * mask_src[:, None, :]).reshape(HG, 3 * H)
    Msrc = jnp.pad(Msrc, ((0, 0), (0, NL - 3 * H)))
    Mdsts = []
    for i in range(3):
        mask_i = (c[None, :] == (i * H + jnp.arange(H)[:, None])).astype(jnp.float32)
        Mi = (att_dst[:, :, None] * mask_i[:, None, :]).reshape(HG, 3 * H)
        Mdsts.append(jnp.pad(Mi, ((0, 0), (0, NL - 3 * H))))
    return Msrc, Mdsts


def _ext_weights(W, att_src, att_dst):
    Msrc, Mdsts = _att_mats(att_src, att_dst)
    WMs = W @ Msrc
    return [jnp.concatenate([W, WMs, W @ Mdsts[i]], axis=1) for i in range(3)]


def _dotT(A, X, dA, dX):
    return jax.lax.dot_general(A, X, (((dA,), (dX,)), ((), ())),
                               preferred_element_type=jnp.float32)


def _attend(he, b_col):
    dl = he[0][HG + NL:, :] + he[1][HG + NL:, :] + he[2][HG + NL:, :]
    ex = []
    for i in range(3):
        L = he[i][HG:HG + NL, :] + dl
        ex.append(jnp.where(L > 0, L, NEG * L))
    m = jnp.maximum(jnp.maximum(ex[0], ex[1]), ex[2])
    ex = [jnp.exp(v - m) for v in ex]
    inv = (1.0 / H) / (ex[0] + ex[1] + ex[2] + 1e-16)
    a1 = ex[1] * inv
    a2 = ex[2] * inv
    hc = [[he[i][k * G:(k + 1) * G, :] for k in range(H)] for i in range(3)]
    d1 = [hc[1][k] - hc[0][k] for k in range(H)]
    d2 = [hc[2][k] - hc[0][k] for k in range(H)]
    s0 = hc[0][0] + hc[0][1] + hc[0][2] + hc[0][3]
    sb = s0 * (1.0 / H) + b_col
    outs = []
    for j in range(3):
        acc = None
        for k in range(H):
            c = j * H + k
            t = a1[c:c + 1, :] * d1[k] + a2[c:c + 1, :] * d2[k]
            acc = t if acc is None else acc + t
        outs.append(acc + sb)
    return outs


def kernel(text_features, audio_features, video_features, type_emb,
           W0, att_src0, att_dst0, b0, W1, att_src1, att_dst1, b1, Wout, bout):
    W0e = _ext_weights(W0, att_src0, att_dst0)
    W1e = _ext_weights(W1, att_src1, att_dst1)
    bout_row = bout.reshape(1, D)
    Wout3 = Wout * (1.0 / 3.0)

    te_pad = [jnp.pad((type_emb[i:i + 1, :] @ W0e[i]).reshape(EXT, 1),
                      ((0, 0), (0, 127))) for i in range(3)]
    b0_pad = jnp.pad(b0.reshape(G, 1), ((0, 0), (0, 127)))
    b1_pad = jnp.pad(b1.reshape(G, 1), ((0, 0), (0, 127)))

    grid = (B // BB,)
    feat_spec = pl.BlockSpec((BB, D), lambda i: (i, 0))
    full = lambda shape: pl.BlockSpec(shape, lambda i: (0,) * len(shape))

    def body(t_ref, a_ref, v_ref,
             W0a, W0b, W0c, te0, te1, te2, b0r,
             W1a, W1b, W1c, b1r, Woutr, boutr, out_ref):
        xs = [t_ref[:], a_ref[:], v_ref[:]]
        W0r = [W0a, W0b, W0c]
        ter = [te0, te1, te2]
        he = [_dotT(W0r[i][:], xs[i], 0, 1) + ter[i][:, 0:1] for i in range(3)]
        ys = [jnp.maximum(y, 0.0) for y in _attend(he, b0r[:, 0:1])]
        W1r = [W1a, W1b, W1c]
        he2 = [_dotT(W1r[i][:], ys[i], 0, 0) for i in range(3)]
        zs = [jnp.maximum(z, 0.0) for z in _attend(he2, b1r[:, 0:1])]
        pooled = zs[0] + zs[1] + zs[2]
        out_ref[:] = _dotT(pooled, Woutr[:], 0, 0) + boutr[:]

    return pl.pallas_call(
        body,
        grid=grid,
        in_specs=[
            feat_spec, feat_spec, feat_spec,
            full((D, EXT)), full((D, EXT)), full((D, EXT)),
            full((EXT, 128)), full((EXT, 128)), full((EXT, 128)), full((G, 128)),
            full((G, EXT)), full((G, EXT)), full((G, EXT)), full((G, 128)),
            full((G, D)), full((1, D)),
        ],
        out_specs=pl.BlockSpec((BB, D), lambda i: (i, 0)),
        out_shape=jax.ShapeDtypeStruct((B, D), jnp.float32),
        compiler_params=pltpu.CompilerParams(
            dimension_semantics=("parallel",)),
    )(text_features, audio_features, video_features,
      W0e[0], W0e[1], W0e[2], te_pad[0], te_pad[1], te_pad[2], b0_pad,
      W1e[0], W1e[1], W1e[2], b1_pad,
      Wout3, bout_row)


# speedup vs baseline: 1.4374x; 1.1857x over previous
import jax
import jax.numpy as jnp
from jax.experimental import pallas as pl
from jax.experimental.pallas import tpu as pltpu

B = 8192
D = 128
G = 128
H = 4
NEG = 0.2

BB = 1024
HG = H * G
NL = 16


def _dotT(A, X, dA, dX):
    return jax.lax.dot_general(A, X, (((dA,), (dX,)), ((), ())),
                               preferred_element_type=jnp.float32)


def _mask_src():
    r = jax.lax.broadcasted_iota(jnp.int32, (HG, NL), 0) // G
    c = jax.lax.broadcasted_iota(jnp.int32, (HG, NL), 1)
    return jnp.where((c % H == r) & (c < 3 * H), 1.0, 0.0)


def _mask_dst():
    r = jax.lax.broadcasted_iota(jnp.int32, (HG, 3 * NL), 0) // G
    c = jax.lax.broadcasted_iota(jnp.int32, (HG, 3 * NL), 1)
    blk = c // NL
    cc = c % NL
    return jnp.where((cc == blk * H + r) & (cc < 3 * H), 1.0, 0.0)


def _attend(h, lg, b_col):
    dl = lg[0][NL:, :] + lg[1][NL:, :] + lg[2][NL:, :]
    ex = []
    for i in range(3):
        L = lg[i][0:NL, :] + dl
        ex.append(jnp.where(L > 0, L, NEG * L))
    m = jnp.maximum(jnp.maximum(ex[0], ex[1]), ex[2])
    ex = [jnp.exp(v - m) for v in ex]
    inv = (1.0 / H) / (ex[0] + ex[1] + ex[2] + 1e-16)
    a1 = ex[1] * inv
    a2 = ex[2] * inv
    hc = [[h[i][k * G:(k + 1) * G, :] for k in range(H)] for i in range(3)]
    d1 = [hc[1][k] - hc[0][k] for k in range(H)]
    d2 = [hc[2][k] - hc[0][k] for k in range(H)]
    s0 = hc[0][0] + hc[0][1] + hc[0][2] + hc[0][3]
    sb = s0 * (1.0 / H) + b_col
    outs = []
    for j in range(3):
        acc = None
        for k in range(H):
            c = j * H + k
            t = a1[c:c + 1, :] * d1[k] + a2[c:c + 1, :] * d2[k]
            acc = t if acc is None else acc + t
        outs.append(acc + sb)
    return outs


def kernel(text_features, audio_features, video_features, type_emb,
           W0, att_src0, att_dst0, b0, W1, att_src1, att_dst1, b1, Wout, bout):
    as0 = att_src0.reshape(HG, 1)
    ad0 = att_dst0.reshape(HG, 1)
    as1 = att_src1.reshape(HG, 1)
    ad1 = att_dst1.reshape(HG, 1)
    b0r = b0.reshape(1, G)
    b1r = b1.reshape(1, G)
    boutr = bout.reshape(1, D)

    grid = (B // BB,)
    feat_spec = pl.BlockSpec((BB, D), lambda i: (i, 0))
    full = lambda shape: pl.BlockSpec(shape, lambda i: (0,) * len(shape))

    def body(t_ref, a_ref, v_ref, te_ref,
             W0_ref, as0_ref, ad0_ref, b0_ref,
             W1_ref, as1_ref, ad1_ref, b1_ref,
             Wout_ref, bout_ref, out_ref,
             lg0_ref, lg1_ref, bc_ref):
        @pl.when(pl.program_id(0) == 0)
        def _prep():
            msrc = _mask_src()
            mdst = _mask_dst()
            for (W_ref, as_ref, ad_ref, lg_ref) in (
                    (W0_ref, as0_ref, ad0_ref, lg0_ref),
                    (W1_ref, as1_ref, ad1_ref, lg1_ref)):
                Ws = jnp.dot(W_ref[:], msrc * as_ref[:],
                             preferred_element_type=jnp.float32)
                Wd = jnp.dot(W_ref[:], mdst * ad_ref[:],
                             preferred_element_type=jnp.float32)
                for i in range(3):
                    lg_ref[:, 2 * NL * i:2 * NL * i + NL] = Ws
                    lg_ref[:, 2 * NL * i + NL:2 * NL * (i + 1)] = (
                        Wd[:, NL * i:NL * (i + 1)])
            eye = jnp.where(
                jax.lax.broadcasted_iota(jnp.int32, (G, G), 0)
                == jax.lax.broadcasted_iota(jnp.int32, (G, G), 1), 1.0, 0.0)
            bc_ref[:, 0:1] = _dotT(eye, b0_ref[:], 0, 1)
            bc_ref[:, 1:2] = _dotT(eye, b1_ref[:], 0, 1)

        xs = [t_ref[:] + te_ref[0:1, :],
              a_ref[:] + te_ref[1:2, :],
              v_ref[:] + te_ref[2:3, :]]
        h1 = [_dotT(W0_ref[:], xs[i], 0, 1) for i in range(3)]
        lgs1 = [_dotT(lg0_ref[:, 2 * NL * i:2 * NL * (i + 1)], xs[i], 0, 1)
                for i in range(3)]
        ys = [jnp.maximum(y, 0.0) for y in _attend(h1, lgs1, bc_ref[:, 0:1])]
        h2 = [_dotT(W1_ref[:], ys[i], 0, 0) for i in range(3)]
        lgs2 = [_dotT(lg1_ref[:, 2 * NL * i:2 * NL * (i + 1)], ys[i], 0, 0)
                for i in range(3)]
        zs = [jnp.maximum(z, 0.0) for z in _attend(h2, lgs2, bc_ref[:, 1:2])]
        pooled = (zs[0] + zs[1] + zs[2]) * (1.0 / 3.0)
        out_ref[:] = _dotT(pooled, Wout_ref[:], 0, 0) + bout_ref[:]

    return pl.pallas_call(
        body,
        grid=grid,
        in_specs=[
            feat_spec, feat_spec, feat_spec,
            full((3, D)),
            full((D, HG)), full((HG, 1)), full((HG, 1)), full((1, G)),
            full((G, HG)), full((HG, 1)), full((HG, 1)), full((1, G)),
            full((G, D)), full((1, D)),
        ],
        out_specs=pl.BlockSpec((BB, D), lambda i: (i, 0)),
        out_shape=jax.ShapeDtypeStruct((B, D), jnp.float32),
        scratch_shapes=[
            pltpu.VMEM((D, 6 * NL), jnp.float32),
            pltpu.VMEM((G, 6 * NL), jnp.float32),
            pltpu.VMEM((G, 128), jnp.float32),
        ],
        compiler_params=pltpu.CompilerParams(
            dimension_semantics=("arbitrary",)),
    )(text_features, audio_features, video_features, type_emb,
      W0, as0, ad0, b0r,
      W1, as1, ad1, b1r,
      Wout, boutr)


# speedup vs baseline: 1.4708x; 1.0232x over previous
import jax
import jax.numpy as jnp
from jax.experimental import pallas as pl
from jax.experimental.pallas import tpu as pltpu

B = 8192
D = 128
G = 128
H = 4
NEG = 0.2

BB = 2048
HG = H * G
NL = 16


def _dotT(A, X, dA, dX):
    return jax.lax.dot_general(A, X, (((dA,), (dX,)), ((), ())),
                               preferred_element_type=jnp.float32)


def _mask_src():
    r = jax.lax.broadcasted_iota(jnp.int32, (HG, NL), 0) // G
    c = jax.lax.broadcasted_iota(jnp.int32, (HG, NL), 1)
    return jnp.where((c % H == r) & (c < 3 * H), 1.0, 0.0)


def _mask_dst():
    r = jax.lax.broadcasted_iota(jnp.int32, (HG, 3 * NL), 0) // G
    c = jax.lax.broadcasted_iota(jnp.int32, (HG, 3 * NL), 1)
    blk = c // NL
    cc = c % NL
    return jnp.where((cc == blk * H + r) & (cc < 3 * H), 1.0, 0.0)


def _attend(h, lg, b_col):
    dl = lg[0][NL:, :] + lg[1][NL:, :] + lg[2][NL:, :]
    ex = []
    for i in range(3):
        L = lg[i][0:NL, :] + dl
        ex.append(jnp.where(L > 0, L, NEG * L))
    m = jnp.maximum(jnp.maximum(ex[0], ex[1]), ex[2])
    ex = [jnp.exp(v - m) for v in ex]
    inv = (1.0 / H) / (ex[0] + ex[1] + ex[2] + 1e-16)
    a1 = ex[1] * inv
    a2 = ex[2] * inv
    hc = [[h[i][k * G:(k + 1) * G, :] for k in range(H)] for i in range(3)]
    d1 = [hc[1][k] - hc[0][k] for k in range(H)]
    d2 = [hc[2][k] - hc[0][k] for k in range(H)]
    s0 = hc[0][0] + hc[0][1] + hc[0][2] + hc[0][3]
    sb = s0 * (1.0 / H) + b_col
    outs = []
    for j in range(3):
        acc = None
        for k in range(H):
            c = j * H + k
            t = a1[c:c + 1, :] * d1[k] + a2[c:c + 1, :] * d2[k]
            acc = t if acc is None else acc + t
        outs.append(acc + sb)
    return outs


def kernel(text_features, audio_features, video_features, type_emb,
           W0, att_src0, att_dst0, b0, W1, att_src1, att_dst1, b1, Wout, bout):
    as0 = att_src0.reshape(HG, 1)
    ad0 = att_dst0.reshape(HG, 1)
    as1 = att_src1.reshape(HG, 1)
    ad1 = att_dst1.reshape(HG, 1)
    b0r = b0.reshape(1, G)
    b1r = b1.reshape(1, G)
    boutr = bout.reshape(1, D)

    grid = (B // BB,)
    feat_spec = pl.BlockSpec((BB, D), lambda i: (i, 0))
    full = lambda shape: pl.BlockSpec(shape, lambda i: (0,) * len(shape))

    def body(t_ref, a_ref, v_ref, te_ref,
             W0_ref, as0_ref, ad0_ref, b0_ref,
             W1_ref, as1_ref, ad1_ref, b1_ref,
             Wout_ref, bout_ref, out_ref,
             lg0_ref, lg1_ref, bc_ref):
        @pl.when(pl.program_id(0) == 0)
        def _prep():
            msrc = _mask_src()
            mdst = _mask_dst()
            for (W_ref, as_ref, ad_ref, lg_ref) in (
                    (W0_ref, as0_ref, ad0_ref, lg0_ref),
                    (W1_ref, as1_ref, ad1_ref, lg1_ref)):
                Ws = jnp.dot(W_ref[:], msrc * as_ref[:],
                             preferred_element_type=jnp.float32)
                Wd = jnp.dot(W_ref[:], mdst * ad_ref[:],
                             preferred_element_type=jnp.float32)
                for i in range(3):
                    lg_ref[:, 2 * NL * i:2 * NL * i + NL] = Ws
                    lg_ref[:, 2 * NL * i + NL:2 * NL * (i + 1)] = (
                        Wd[:, NL * i:NL * (i + 1)])
            eye = jnp.where(
                jax.lax.broadcasted_iota(jnp.int32, (G, G), 0)
                == jax.lax.broadcasted_iota(jnp.int32, (G, G), 1), 1.0, 0.0)
            bc_ref[:, 0:1] = _dotT(eye, b0_ref[:], 0, 1)
            bc_ref[:, 1:2] = _dotT(eye, b1_ref[:], 0, 1)

        xs = [t_ref[:] + te_ref[0:1, :],
              a_ref[:] + te_ref[1:2, :],
              v_ref[:] + te_ref[2:3, :]]
        h1 = [_dotT(W0_ref[:], xs[i], 0, 1) for i in range(3)]
        lgs1 = [_dotT(lg0_ref[:, 2 * NL * i:2 * NL * (i + 1)], xs[i], 0, 1)
                for i in range(3)]
        ys = [jnp.maximum(y, 0.0) for y in _attend(h1, lgs1, bc_ref[:, 0:1])]
        h2 = [_dotT(W1_ref[:], ys[i], 0, 0) for i in range(3)]
        lgs2 = [_dotT(lg1_ref[:, 2 * NL * i:2 * NL * (i + 1)], ys[i], 0, 0)
                for i in range(3)]
        zs = [jnp.maximum(z, 0.0) for z in _attend(h2, lgs2, bc_ref[:, 1:2])]
        pooled = (zs[0] + zs[1] + zs[2]) * (1.0 / 3.0)
        out_ref[:] = _dotT(pooled, Wout_ref[:], 0, 0) + bout_ref[:]

    return pl.pallas_call(
        body,
        grid=grid,
        in_specs=[
            feat_spec, feat_spec, feat_spec,
            full((3, D)),
            full((D, HG)), full((HG, 1)), full((HG, 1)), full((1, G)),
            full((G, HG)), full((HG, 1)), full((HG, 1)), full((1, G)),
            full((G, D)), full((1, D)),
        ],
        out_specs=pl.BlockSpec((BB, D), lambda i: (i, 0)),
        out_shape=jax.ShapeDtypeStruct((B, D), jnp.float32),
        scratch_shapes=[
            pltpu.VMEM((D, 6 * NL), jnp.float32),
            pltpu.VMEM((G, 6 * NL), jnp.float32),
            pltpu.VMEM((G, 128), jnp.float32),
        ],
        compiler_params=pltpu.CompilerParams(
            dimension_semantics=("arbitrary",)),
    )(text_features, audio_features, video_features, type_emb,
      W0, as0, ad0, b0r,
      W1, as1, ad1, b1r,
      Wout, boutr)
